# baseline (speedup 1.0000x reference)
import functools

import jax
import jax.numpy as jnp
from jax.experimental import pallas as pl

_ROTATIONS = (13, 15, 26, 6, 17, 29, 16, 24)
_TINY = 1.1754943508222875e-38


def _threefry_bits(flat):
    ks0 = jnp.uint32(0)
    ks1 = jnp.uint32(42)
    ks2 = ks0 ^ ks1 ^ jnp.uint32(0x1BD11BDA)
    ks = (ks0, ks1, ks2)
    x0 = jnp.full_like(flat, ks0)
    x1 = flat + ks1
    for g in range(5):
        rots = _ROTATIONS[0:4] if g % 2 == 0 else _ROTATIONS[4:8]
        for r in rots:
            x0 = x0 + x1
            x1 = ((x1 << jnp.uint32(r)) | (x1 >> jnp.uint32(32 - r))) ^ x0
        x0 = x0 + ks[(g + 1) % 3]
        x1 = x1 + ks[(g + 2) % 3] + jnp.uint32(g + 1)
    return x0 ^ x1


def _sample_block(p_ref, sel_ref, psel_ref, *, ncols, strip_rows):
    i = pl.program_id(0)
    block_rows = p_ref.shape[0]
    nstrips = block_rows // strip_rows

    col = jax.lax.broadcasted_iota(jnp.int32, (strip_rows, ncols), 1)
    flat0 = (
        jax.lax.broadcasted_iota(jnp.int32, (strip_rows, ncols), 0) * ncols + col
    ).astype(jnp.uint32)
    block_base = i * (block_rows * ncols)

    def strip(j, _):
        p = p_ref[pl.ds(j * strip_rows, strip_rows), :]
        flat = flat0 + (block_base + j * (strip_rows * ncols)).astype(jnp.uint32)

        bits = _threefry_bits(flat)
        fbits = (bits >> jnp.uint32(9)) | jnp.uint32(0x3F800000)
        frac = jax.lax.bitcast_convert_type(fbits, jnp.float32) - jnp.float32(1.0)
        u = jnp.maximum(frac, _TINY)
        gumbel = -jnp.log(-jnp.log(u))
        val = gumbel + jnp.log(p)

        sel = jnp.argmax(val, axis=1).astype(jnp.int32)
        sel_ref[pl.ds(j * strip_rows, strip_rows), :] = sel[:, None]
        psel = jnp.max(jnp.where(col == sel[:, None], p, jnp.float32(0.0)), axis=1)
        psel_ref[pl.ds(j * strip_rows, strip_rows), :] = psel[:, None]

    jax.lax.fori_loop(0, nstrips, strip, None)


def kernel(probs):
    nrows, ncols = probs.shape
    block_rows = 512
    strip_rows = 16
    if nrows % block_rows:
        block_rows = nrows
        strip_rows = nrows
    grid = nrows // block_rows
    sel, psel = pl.pallas_call(
        functools.partial(_sample_block, ncols=ncols, strip_rows=strip_rows),
        grid=(grid,),
        in_specs=[pl.BlockSpec((block_rows, ncols), lambda i: (i, 0))],
        out_specs=[
            pl.BlockSpec((block_rows, 1), lambda i: (i, 0)),
            pl.BlockSpec((block_rows, 1), lambda i: (i, 0)),
        ],
        out_shape=[
            jax.ShapeDtypeStruct((nrows, 1), jnp.int32),
            jax.ShapeDtypeStruct((nrows, 1), jnp.float32),
        ],
    )(probs)
    return sel[:, 0], psel[:, 0]


# speedup vs baseline: 1.1366x; 1.1366x over previous
import functools

import jax
import jax.numpy as jnp
from jax.experimental import pallas as pl

_ROTATIONS = (13, 15, 26, 6, 17, 29, 16, 24)
_TINY = 1.1754943508222875e-38


def _threefry_bits(flat):
    ks0 = jnp.uint32(0)
    ks1 = jnp.uint32(42)
    ks2 = ks0 ^ ks1 ^ jnp.uint32(0x1BD11BDA)
    ks = (ks0, ks1, ks2)
    x0 = jnp.full_like(flat, ks0)
    x1 = flat + ks1
    for g in range(5):
        rots = _ROTATIONS[0:4] if g % 2 == 0 else _ROTATIONS[4:8]
        for r in rots:
            x0 = x0 + x1
            x1 = ((x1 << jnp.uint32(r)) | (x1 >> jnp.uint32(32 - r))) ^ x0
        x0 = x0 + ks[(g + 1) % 3]
        x1 = x1 + ks[(g + 2) % 3] + jnp.uint32(g + 1)
    return x0 ^ x1


def _sample_block(p_ref, sel_ref, psel_ref, *, ncols, col_chunk):
    i = pl.program_id(0)
    rows = p_ref.shape[0]

    best_v = None
    best_i = None
    best_p = None
    for c0 in range(0, ncols, col_chunk):
        cw = min(col_chunk, ncols - c0)
        p = p_ref[:, pl.ds(c0, cw)]
        col = jax.lax.broadcasted_iota(jnp.int32, (rows, cw), 1)
        row = jax.lax.broadcasted_iota(jnp.int32, (rows, cw), 0) + i * rows
        flat = (row * ncols + (col + c0)).astype(jnp.uint32)

        bits = _threefry_bits(flat)
        fbits = (bits >> jnp.uint32(9)) | jnp.uint32(0x3F800000)
        frac = jax.lax.bitcast_convert_type(fbits, jnp.float32) - jnp.float32(1.0)
        u = jnp.maximum(frac, _TINY)
        gumbel = -jnp.log(-jnp.log(u))
        val = gumbel + jnp.log(p)

        m = jnp.max(val, axis=1, keepdims=True)
        idx_local = jnp.argmax(val, axis=1).astype(jnp.int32)[:, None]
        idx = idx_local + c0
        pm = jnp.max(
            jnp.where(col == idx_local, p, jnp.float32(0.0)), axis=1, keepdims=True
        )
        if best_v is None:
            best_v, best_i, best_p = m, idx, pm
        else:
            upd = m > best_v
            best_v = jnp.where(upd, m, best_v)
            best_i = jnp.where(upd, idx, best_i)
            best_p = jnp.where(upd, pm, best_p)

    sel_ref[...] = best_i
    psel_ref[...] = best_p


def kernel(probs):
    nrows, ncols = probs.shape
    block_rows = 256
    if nrows % block_rows:
        block_rows = nrows
    grid = nrows // block_rows
    sel, psel = pl.pallas_call(
        functools.partial(_sample_block, ncols=ncols, col_chunk=256),
        grid=(grid,),
        in_specs=[pl.BlockSpec((block_rows, ncols), lambda i: (i, 0))],
        out_specs=[
            pl.BlockSpec((block_rows, 1), lambda i: (i, 0)),
            pl.BlockSpec((block_rows, 1), lambda i: (i, 0)),
        ],
        out_shape=[
            jax.ShapeDtypeStruct((nrows, 1), jnp.int32),
            jax.ShapeDtypeStruct((nrows, 1), jnp.float32),
        ],
    )(probs)
    return sel[:, 0], psel[:, 0]


# speedup vs baseline: 1.6472x; 1.4492x over previous
import functools

import jax
import jax.numpy as jnp
from jax.experimental import pallas as pl

_ROTATIONS = (13, 15, 26, 6, 17, 29, 16, 24)
_TINY = 1.1754943508222875e-38


def _threefry_bits(flat):
    ks0 = jnp.uint32(0)
    ks1 = jnp.uint32(42)
    ks2 = ks0 ^ ks1 ^ jnp.uint32(0x1BD11BDA)
    ks = (ks0, ks1, ks2)
    x0 = jnp.full_like(flat, ks0)
    x1 = flat + ks1
    for g in range(5):
        rots = _ROTATIONS[0:4] if g % 2 == 0 else _ROTATIONS[4:8]
        for r in rots:
            x0 = x0 + x1
            rot = x1 * jnp.uint32(1 << r) + (x1 >> jnp.uint32(32 - r))
            x1 = rot ^ x0
        x0 = x0 + ks[(g + 1) % 3]
        x1 = x1 + ks[(g + 2) % 3] + jnp.uint32(g + 1)
    return x0 ^ x1


def _sample_block(p_ref, sel_ref, psel_ref, *, ncols):
    i = pl.program_id(0)
    p = p_ref[...]
    rows, cols = p.shape
    col = jax.lax.broadcasted_iota(jnp.int32, (rows, cols), 1)
    row = jax.lax.broadcasted_iota(jnp.int32, (rows, cols), 0) + i * rows
    flat = (row * ncols + col).astype(jnp.uint32)

    bits = _threefry_bits(flat)
    fbits = (bits >> jnp.uint32(9)) | jnp.uint32(0x3F800000)
    frac = jax.lax.bitcast_convert_type(fbits, jnp.float32) - jnp.float32(1.0)
    u = jnp.maximum(frac, _TINY)
    gumbel = -jnp.log(-jnp.log(u))
    val = gumbel + jnp.log(p)

    sel = jnp.argmax(val, axis=1).astype(jnp.int32)
    sel_ref[...] = sel[:, None]
    psel = jnp.max(jnp.where(col == sel[:, None], p, jnp.float32(0.0)), axis=1)
    psel_ref[...] = psel[:, None]


def kernel(probs):
    nrows, ncols = probs.shape
    block_rows = 1024
    if nrows % block_rows:
        block_rows = nrows
    grid = nrows // block_rows
    sel, psel = pl.pallas_call(
        functools.partial(_sample_block, ncols=ncols),
        grid=(grid,),
        in_specs=[pl.BlockSpec((block_rows, ncols), lambda i: (i, 0))],
        out_specs=[
            pl.BlockSpec((block_rows, 1), lambda i: (i, 0)),
            pl.BlockSpec((block_rows, 1), lambda i: (i, 0)),
        ],
        out_shape=[
            jax.ShapeDtypeStruct((nrows, 1), jnp.int32),
            jax.ShapeDtypeStruct((nrows, 1), jnp.float32),
        ],
    )(probs)
    return sel[:, 0], psel[:, 0]


# speedup vs baseline: 1.6627x; 1.0094x over previous
import functools

import jax
import jax.numpy as jnp
from jax.experimental import pallas as pl

_ROT20 = (13, 15, 26, 6, 17, 29, 16, 24, 13, 15, 26, 6, 17, 29, 16, 24,
          13, 15, 26, 6)
_KS1 = 42
_KS2 = 42 ^ 0x1BD11BDA
_INJ = (
    (_KS1, _KS2 + 1),
    (_KS2, 2),
    (None, _KS1 + 3),
    (_KS1, _KS2 + 4),
    (_KS2, 5),
)
_TINY = 1.1754943508222875e-38


def _threefry_bits(flat42):
    x1 = flat42
    x0 = x1
    x1 = ((x1 << jnp.uint32(13)) | (x1 >> jnp.uint32(19))) ^ x0
    for rnd in range(1, 20):
        r = _ROT20[rnd]
        x0 = x0 + x1
        x1 = ((x1 << jnp.uint32(r)) | (x1 >> jnp.uint32(32 - r))) ^ x0
        if rnd % 4 == 3:
            a, b = _INJ[rnd // 4]
            if a is not None:
                x0 = x0 + jnp.uint32(a)
            x1 = x1 + jnp.uint32(b)
    return x0 ^ x1


def _sample_block(p_ref, sel_ref, psel_ref, *, ncols, strip_rows):
    i = pl.program_id(0)
    rows = p_ref.shape[0]
    nstrips = rows // strip_rows

    col = jax.lax.broadcasted_iota(jnp.int32, (strip_rows, ncols), 1)
    base0 = (
        jax.lax.broadcasted_iota(jnp.int32, (strip_rows, ncols), 0) * ncols
        + col + _KS1
    ).astype(jnp.uint32)

    for s in range(nstrips):
        p = p_ref[pl.ds(s * strip_rows, strip_rows), :]
        flat42 = base0 + jnp.uint32((0 + s) * strip_rows * ncols) + (
            i * (rows * ncols)
        ).astype(jnp.uint32)

        bits = _threefry_bits(flat42)
        fbits = (bits >> jnp.uint32(9)) | jnp.uint32(0x3F800000)
        frac = jax.lax.bitcast_convert_type(fbits, jnp.float32) - jnp.float32(1.0)
        u = jnp.maximum(frac, _TINY)
        gumbel = -jnp.log(-jnp.log(u))
        val = gumbel + jnp.log(p)

        sel = jnp.argmax(val, axis=1).astype(jnp.int32)
        sel_ref[pl.ds(s * strip_rows, strip_rows), :] = sel[:, None]
        psel = jnp.max(jnp.where(col == sel[:, None], p, jnp.float32(0.0)), axis=1)
        psel_ref[pl.ds(s * strip_rows, strip_rows), :] = psel[:, None]


def kernel(probs):
    nrows, ncols = probs.shape
    block_rows = 512
    strip_rows = 32
    if nrows % block_rows:
        block_rows = nrows
        strip_rows = nrows
    grid = nrows // block_rows
    sel, psel = pl.pallas_call(
        functools.partial(_sample_block, ncols=ncols, strip_rows=strip_rows),
        grid=(grid,),
        in_specs=[pl.BlockSpec((block_rows, ncols), lambda i: (i, 0))],
        out_specs=[
            pl.BlockSpec((block_rows, 1), lambda i: (i, 0)),
            pl.BlockSpec((block_rows, 1), lambda i: (i, 0)),
        ],
        out_shape=[
            jax.ShapeDtypeStruct((nrows, 1), jnp.int32),
            jax.ShapeDtypeStruct((nrows, 1), jnp.float32),
        ],
    )(probs)
    return sel[:, 0], psel[:, 0]
